# Initial kernel scaffold; baseline (speedup 1.0000x reference)
import jax
import jax.numpy as jnp
from jax.experimental import pallas as pl


def kernel(predict, label):
    raise NotImplementedError("write your pallas kernel here")


# speedup vs baseline: 1.7218x; 1.7218x over previous
import functools

import jax
import jax.numpy as jnp
from jax import lax
from jax.experimental import pallas as pl
from jax.experimental.pallas import tpu as pltpu
from jax.experimental.pallas import tpu_sc as plsc

_B = 16384
_C = 1000
_NC = 2
_NS = 16
_L = 16
_NW = _NC * _NS
_BPW = _B // _NW
_CHUNK = 128
_NCHUNK = _BPW // _CHUNK

_mesh = plsc.VectorSubcoreMesh(core_axis_name="c", subcore_axis_name="s")


@functools.partial(
    pl.kernel,
    mesh=_mesh,
    out_type=jax.ShapeDtypeStruct((_NW, _L), jnp.float32),
    scratch_types=[
        pltpu.VMEM((_BPW,), jnp.int32),
        pltpu.VMEM((_NCHUNK, _CHUNK), jnp.int32),
        pltpu.VMEM((_NCHUNK, _CHUNK), jnp.float32),
        pltpu.VMEM((_L,), jnp.float32),
        pltpu.SemaphoreType.DMA,
    ],
)
def _gather_partial_sums(pred_hbm, lab_hbm, out_hbm, lab_v, idx_v, val_v,
                         acc_v, sem):
    wid = lax.axis_index("s") * _NC + lax.axis_index("c")
    base = wid * _BPW
    pltpu.sync_copy(lab_hbm.at[pl.ds(base, _BPW)], lab_v)

    iota = lax.iota(jnp.int32, _L)
    per_chunk = _CHUNK // _L
    for j in range(_BPW // _L):
        lv = lab_v[pl.ds(j * _L, _L)]
        rows = base + j * _L + iota
        idx_v[j // per_chunk, pl.ds((j % per_chunk) * _L, _L)] = rows * _C + lv

    copies = [
        pltpu.async_copy(pred_hbm.at[idx_v.at[c]], val_v.at[c], sem)
        for c in range(_NCHUNK)
    ]
    for cp in copies:
        cp.wait()

    acc = jnp.zeros((_L,), jnp.float32)
    for j in range(_BPW // _L):
        acc = acc + val_v[j // per_chunk, pl.ds((j % per_chunk) * _L, _L)]
    acc_v[...] = acc
    pltpu.sync_copy(acc_v, out_hbm.at[wid])


def kernel(predict, label):
    pred_flat = predict.reshape(-1)
    partial = _gather_partial_sums(pred_flat, label.astype(jnp.int32))
    return partial.sum() / predict.shape[0]


# speedup vs baseline: 2.6085x; 1.5149x over previous
import functools

import jax
import jax.numpy as jnp
from jax import lax
from jax.experimental import pallas as pl
from jax.experimental.pallas import tpu as pltpu
from jax.experimental.pallas import tpu_sc as plsc

_B = 16384
_C = 1000
_BLK = 512
_NBLK = _B // _BLK


def _tc_body(lab_ref, pred_ref, out_ref, acc_ref):
    i = pl.program_id(0)
    lab = lab_ref[0, 0, :]
    cols = lax.broadcasted_iota(jnp.int32, (_BLK, _C), 1)
    sel = cols == lab[:, None]
    part = jnp.sum(jnp.where(sel, pred_ref[...], 0.0))

    @pl.when(i == 0)
    def _():
        acc_ref[0] = 0.0

    acc_ref[0] += part

    @pl.when(i == _NBLK - 1)
    def _():
        out_ref[0, 0] = acc_ref[0]


_tc_call = pl.pallas_call(
    _tc_body,
    grid=(_NBLK,),
    in_specs=[
        pl.BlockSpec((1, 1, _BLK), lambda i: (i, 0, 0)),
        pl.BlockSpec((_BLK, _C), lambda i: (i, 0)),
    ],
    out_specs=pl.BlockSpec(memory_space=pltpu.SMEM),
    out_shape=jax.ShapeDtypeStruct((1, 1), jnp.float32),
    scratch_shapes=[pltpu.SMEM((1,), jnp.float32)],
)


def kernel(predict, label):
    lab3 = label.astype(jnp.int32).reshape(_NBLK, 1, _BLK)
    total = _tc_call(lab3, predict)
    return total[0, 0] / predict.shape[0]
